# baseline (speedup 1.0000x reference)
import functools

import jax
import jax.numpy as jnp
from jax import lax
from jax.experimental import pallas as pl
from jax.experimental.pallas import tpu as pltpu
from jax.experimental.pallas import tpu_sc as plsc

N = 10000
E = 320000
F = 128

NUM_SC = 2
NUM_TILES = 16
NW = NUM_SC * NUM_TILES
EDGES_PER_W = E // NW
CHUNK = 80
NCHUNKS = EDGES_PER_W // CHUNK
NPAD = 10240
ROWS_PER_TILE = NPAD // NUM_TILES


def _sc_segment_sum(x, edges_flat, zeros_blk):
    mesh = plsc.VectorSubcoreMesh(core_axis_name="c", subcore_axis_name="s")

    @functools.partial(
        pl.kernel,
        out_type=[jax.ShapeDtypeStruct((NPAD, F), jnp.float32),
                  jax.ShapeDtypeStruct((NPAD, F), jnp.float32)],
        mesh=mesh,
        scratch_types=(
            [pltpu.VMEM((CHUNK,), jnp.int32)] * 6
            + [pltpu.VMEM((CHUNK, F), jnp.float32)] * 3
            + [pltpu.VMEM_SHARED((NPAD, F), jnp.float32)]
            + [pltpu.SemaphoreType.DMA] * 12
        ),
    )
    def seg_sum(x_hbm, e_hbm, z_hbm, out0_hbm, out1_hbm,
                idx_s0, idx_s1, idx_s2, idx_d0, idx_d1, idx_d2,
                rows0, rows1, rows2, acc,
                isem0, isem1, isem2, dsem0, dsem1, dsem2,
                gsem0, gsem1, gsem2, ssem0, ssem1, ssem2):
        c = lax.axis_index("c")
        s = lax.axis_index("s")
        wid = c * NUM_TILES + s
        idx_s = (idx_s0, idx_s1, idx_s2)
        idx_d = (idx_d0, idx_d1, idx_d2)
        rows = (rows0, rows1, rows2)
        isem = (isem0, isem1, isem2)
        dsem = (dsem0, dsem1, dsem2)
        gsem = (gsem0, gsem1, gsem2)
        ssem = (ssem0, ssem1, ssem2)

        row0 = s * ROWS_PER_TILE
        pltpu.sync_copy(z_hbm, rows0)
        for t in range(ROWS_PER_TILE // CHUNK):
            pltpu.sync_copy(rows0, acc.at[pl.ds(row0 + t * CHUNK, CHUNK)])
        plsc.subcore_barrier()

        ebase = wid * EDGES_PER_W

        def start_src(i, b):
            pltpu.async_copy(e_hbm.at[pl.ds(ebase + i * CHUNK, CHUNK)],
                             idx_s[b], isem[b])

        def start_dst(i, b):
            pltpu.async_copy(e_hbm.at[pl.ds(E + ebase + i * CHUNK, CHUNK)],
                             idx_d[b], dsem[b])

        def wait_src(b):
            pltpu.make_async_copy(e_hbm.at[pl.ds(0, CHUNK)], idx_s[b], isem[b]).wait()

        def wait_dst(b):
            pltpu.make_async_copy(e_hbm.at[pl.ds(0, CHUNK)], idx_d[b], dsem[b]).wait()

        def start_gather(b):
            pltpu.async_copy(x_hbm.at[idx_s[b]], rows[b], gsem[b])

        def wait_gather(b):
            pltpu.make_async_copy(x_hbm.at[idx_s[b]], rows[b], gsem[b]).wait()

        def start_scatter(b):
            pltpu.async_copy(rows[b], acc.at[idx_d[b]], ssem[b], add=True)

        def wait_scatter(b):
            pltpu.make_async_copy(rows[b], acc.at[idx_d[b]], ssem[b]).wait()

        def steady(i, b, k_pred=None, skip_c_wait=False):
            bp = (b + 2) % 3
            if not skip_c_wait:
                wait_scatter(b)
            start_dst(i, b)
            wait_src(b)
            start_gather(b)
            wait_gather(bp)
            if k_pred is None:
                start_src(i + 2, bp)
            else:
                @pl.when(k_pred)
                def _():
                    start_src(i + 2, bp)
            wait_dst(bp)
            start_scatter(bp)

        start_src(0, 0)
        start_dst(0, 0)
        start_src(1, 1)
        start_dst(1, 1)
        start_src(2, 2)
        wait_src(0)
        start_gather(0)
        wait_src(1)
        start_gather(1)
        wait_gather(0)
        start_src(3, 0)
        wait_dst(0)
        start_scatter(0)

        steady(2, 2, skip_c_wait=True)
        steady(3, 0)
        steady(4, 1)

        def body(k, carry):
            i0 = 5 + 3 * k
            steady(i0, 2)
            steady(i0 + 1, 0, k_pred=(k < (NCHUNKS - 5) // 3 - 1))
            steady(i0 + 2, 1, k_pred=(k < (NCHUNKS - 5) // 3 - 1))
            return carry

        lax.fori_loop(0, (NCHUNKS - 5) // 3, body, 0, unroll=False)

        wait_gather(1)
        wait_dst(1)
        start_scatter(1)
        wait_scatter(2)
        wait_scatter(0)
        wait_scatter(1)

        plsc.subcore_barrier()
        stripe = pl.ds(row0, ROWS_PER_TILE)

        @pl.when(c == 0)
        def _():
            pltpu.sync_copy(acc.at[stripe], out0_hbm.at[stripe])

        @pl.when(c == 1)
        def _():
            pltpu.sync_copy(acc.at[stripe], out1_hbm.at[stripe])

    return seg_sum(x, edges_flat, zeros_blk)


_ROW_SPEC = pl.BlockSpec((1000, F), lambda i: (i, 0))
_W_SPEC = pl.BlockSpec((F, F), lambda i: (0, 0))
_B_SPEC = pl.BlockSpec((1, F), lambda i: (0, 0))
_GRID = (N // 1000,)


def _tc_self_mm(x, w_self):
    def body(x_r, w_r, o_r):
        o_r[...] = jnp.dot(x_r[...], w_r[...], preferred_element_type=jnp.float32)

    return pl.pallas_call(
        body,
        grid=_GRID,
        in_specs=[_ROW_SPEC, _W_SPEC],
        out_specs=_ROW_SPEC,
        out_shape=jax.ShapeDtypeStruct((N, F), jnp.float32),
    )(x, w_self)


def _tc_combine(xw, p0, p1, w_nbr, b, residual=None):
    def body(*refs):
        if residual is None:
            xw_r, p0_r, p1_r, wn_r, b_r, o_r = refs
        else:
            xw_r, p0_r, p1_r, wn_r, b_r, res_r, o_r = refs
        agg = p0_r[...] + p1_r[...]
        acc = xw_r[...] + jnp.dot(agg, wn_r[...], preferred_element_type=jnp.float32)
        acc = jnp.maximum(acc + b_r[...], 0.0)
        if residual is not None:
            acc = acc + res_r[...]
        o_r[...] = acc

    in_specs = [_ROW_SPEC, _ROW_SPEC, _ROW_SPEC, _W_SPEC, _B_SPEC]
    args = [xw, p0, p1, w_nbr, b.reshape(1, F)]
    if residual is not None:
        in_specs.append(_ROW_SPEC)
        args.append(residual)

    return pl.pallas_call(
        body,
        grid=_GRID,
        in_specs=in_specs,
        out_specs=_ROW_SPEC,
        out_shape=jax.ShapeDtypeStruct((N, F), jnp.float32),
    )(*args)


def kernel(lv, edge_index1, edge_index2, W_self1, W_nbr1, b1, W_self2, W_nbr2, b2):
    e1 = edge_index1.reshape(2 * E)
    e2 = edge_index2.reshape(2 * E)
    zeros_blk = jnp.zeros((CHUNK, F), jnp.float32)

    p0, p1 = _sc_segment_sum(lv, e1, zeros_blk)
    xw1 = _tc_self_mm(lv, W_self1)
    h = _tc_combine(xw1, p0, p1, W_nbr1, b1)
    q0, q1 = _sc_segment_sum(h, e2, zeros_blk)
    xw2 = _tc_self_mm(h, W_self2)
    out = _tc_combine(xw2, q0, q1, W_nbr2, b2, residual=lv)
    return out


# speedup vs baseline: 1.0328x; 1.0328x over previous
import functools

import jax
import jax.numpy as jnp
from jax import lax
from jax.experimental import pallas as pl
from jax.experimental.pallas import tpu as pltpu
from jax.experimental.pallas import tpu_sc as plsc

N = 10000
E = 320000
F = 128

NUM_SC = 2
NUM_TILES = 16
NW = NUM_SC * NUM_TILES
EDGES_PER_W = E // NW
CHUNK = 80
NCHUNKS = EDGES_PER_W // CHUNK
NPAD = 10240
ROWS_PER_TILE = NPAD // NUM_TILES


def _sc_segment_sum(x, edges_flat, zeros_blk):
    mesh = plsc.VectorSubcoreMesh(core_axis_name="c", subcore_axis_name="s")

    @functools.partial(
        pl.kernel,
        out_type=[jax.ShapeDtypeStruct((NPAD, F), jnp.float32),
                  jax.ShapeDtypeStruct((NPAD, F), jnp.float32)],
        mesh=mesh,
        scratch_types=(
            [pltpu.VMEM((CHUNK,), jnp.int32)] * 6
            + [pltpu.VMEM((CHUNK, F), jnp.float32)] * 3
            + [pltpu.VMEM_SHARED((NPAD, F), jnp.float32)]
            + [pltpu.SemaphoreType.DMA] * 12
        ),
    )
    def seg_sum(x_hbm, e_hbm, z_hbm, out0_hbm, out1_hbm,
                idx_s0, idx_s1, idx_s2, idx_d0, idx_d1, idx_d2,
                rows0, rows1, rows2, acc,
                isem0, isem1, isem2, dsem0, dsem1, dsem2,
                gsem0, gsem1, gsem2, ssem0, ssem1, ssem2):
        c = lax.axis_index("c")
        s = lax.axis_index("s")
        wid = c * NUM_TILES + s
        idx_s = (idx_s0, idx_s1, idx_s2)
        idx_d = (idx_d0, idx_d1, idx_d2)
        rows = (rows0, rows1, rows2)
        isem = (isem0, isem1, isem2)
        dsem = (dsem0, dsem1, dsem2)
        gsem = (gsem0, gsem1, gsem2)
        ssem = (ssem0, ssem1, ssem2)

        row0 = s * ROWS_PER_TILE
        pltpu.sync_copy(z_hbm, rows0)
        for t in range(ROWS_PER_TILE // CHUNK):
            pltpu.sync_copy(rows0, acc.at[pl.ds(row0 + t * CHUNK, CHUNK)])
        plsc.subcore_barrier()

        ebase = wid * EDGES_PER_W

        def start_src(i, b):
            pltpu.async_copy(e_hbm.at[pl.ds(ebase + i * CHUNK, CHUNK)],
                             idx_s[b], isem[b])

        def start_dst(i, b):
            pltpu.async_copy(e_hbm.at[pl.ds(E + ebase + i * CHUNK, CHUNK)],
                             idx_d[b], dsem[b])

        def wait_src(b):
            pltpu.make_async_copy(e_hbm.at[pl.ds(0, CHUNK)], idx_s[b], isem[b]).wait()

        def wait_dst(b):
            pltpu.make_async_copy(e_hbm.at[pl.ds(0, CHUNK)], idx_d[b], dsem[b]).wait()

        def start_gather(b):
            pltpu.async_copy(x_hbm.at[idx_s[b]], rows[b], gsem[b])

        def wait_gather(b):
            pltpu.make_async_copy(x_hbm.at[idx_s[b]], rows[b], gsem[b]).wait()

        def start_scatter(b):
            pltpu.async_copy(rows[b], acc.at[idx_d[b]], ssem[b], add=True)

        def wait_scatter(b):
            pltpu.make_async_copy(rows[b], acc.at[idx_d[b]], ssem[b]).wait()

        def steady(i, b, k_pred=None, skip_c_wait=False):
            bp = (b + 2) % 3
            if not skip_c_wait:
                wait_scatter(b)
            start_dst(i, b)
            wait_src(b)
            start_gather(b)
            wait_gather(bp)
            if k_pred is None:
                start_src(i + 2, bp)
            else:
                @pl.when(k_pred)
                def _():
                    start_src(i + 2, bp)
            wait_dst(bp)
            start_scatter(bp)

        start_src(0, 0)
        start_dst(0, 0)
        start_src(1, 1)
        start_dst(1, 1)
        start_src(2, 2)
        wait_src(0)
        start_gather(0)
        wait_src(1)
        start_gather(1)
        wait_gather(0)
        start_src(3, 0)
        wait_dst(0)
        start_scatter(0)

        steady(2, 2, skip_c_wait=True)
        steady(3, 0)
        steady(4, 1)

        def body(k, carry):
            i0 = 5 + 3 * k
            steady(i0, 2)
            steady(i0 + 1, 0, k_pred=(k < (NCHUNKS - 5) // 3 - 1))
            steady(i0 + 2, 1, k_pred=(k < (NCHUNKS - 5) // 3 - 1))
            return carry

        lax.fori_loop(0, (NCHUNKS - 5) // 3, body, 0, unroll=False)

        wait_gather(1)
        wait_dst(1)
        start_scatter(1)
        wait_scatter(2)
        wait_scatter(0)
        wait_scatter(1)

        plsc.subcore_barrier()
        stripe = pl.ds(row0, ROWS_PER_TILE)

        @pl.when(c == 0)
        def _():
            pltpu.sync_copy(acc.at[stripe], out0_hbm.at[stripe])

        @pl.when(c == 1)
        def _():
            pltpu.sync_copy(acc.at[stripe], out1_hbm.at[stripe])

    return seg_sum(x, edges_flat, zeros_blk)


_TC_BLK = 2000
_ROW_SPEC = pl.BlockSpec((_TC_BLK, F), lambda i: (i, 0))
_W_SPEC = pl.BlockSpec((F, F), lambda i: (0, 0))
_B_SPEC = pl.BlockSpec((1, F), lambda i: (0, 0))
_GRID = (N // _TC_BLK,)


def _tc_layer(x, p0, p1, w_self, w_nbr, b, residual=None):
    def body(*refs):
        if residual is None:
            x_r, p0_r, p1_r, ws_r, wn_r, b_r, o_r = refs
        else:
            x_r, p0_r, p1_r, ws_r, wn_r, b_r, res_r, o_r = refs
        agg = p0_r[...] + p1_r[...]
        acc = jnp.dot(x_r[...], ws_r[...], preferred_element_type=jnp.float32)
        acc = acc + jnp.dot(agg, wn_r[...], preferred_element_type=jnp.float32)
        acc = jnp.maximum(acc + b_r[...], 0.0)
        if residual is not None:
            acc = acc + res_r[...]
        o_r[...] = acc

    in_specs = [_ROW_SPEC, _ROW_SPEC, _ROW_SPEC, _W_SPEC, _W_SPEC, _B_SPEC]
    args = [x, p0, p1, w_self, w_nbr, b.reshape(1, F)]
    if residual is not None:
        in_specs.append(_ROW_SPEC)
        args.append(residual)

    return pl.pallas_call(
        body,
        grid=_GRID,
        in_specs=in_specs,
        out_specs=_ROW_SPEC,
        out_shape=jax.ShapeDtypeStruct((N, F), jnp.float32),
    )(*args)


def kernel(lv, edge_index1, edge_index2, W_self1, W_nbr1, b1, W_self2, W_nbr2, b2):
    e1 = edge_index1.reshape(2 * E)
    e2 = edge_index2.reshape(2 * E)
    zeros_blk = jnp.zeros((CHUNK, F), jnp.float32)

    p0, p1 = _sc_segment_sum(lv, e1, zeros_blk)
    h = _tc_layer(lv, p0, p1, W_self1, W_nbr1, b1)
    q0, q1 = _sc_segment_sum(h, e2, zeros_blk)
    out = _tc_layer(h, q0, q1, W_self2, W_nbr2, b2, residual=lv)
    return out
